# Initial kernel scaffold; baseline (speedup 1.0000x reference)
import jax
import jax.numpy as jnp
from jax.experimental import pallas as pl


def kernel(x, edge_index, W, b):
    raise NotImplementedError("write your pallas kernel here")


# speedup vs baseline: 147.3359x; 147.3359x over previous
import dataclasses
import functools

import jax
import jax.numpy as jnp
from jax import lax
from jax.experimental import pallas as pl
from jax.experimental.pallas import tpu as pltpu
from jax.experimental.pallas import tpu_sc as plsc

N = 2000
F = 128
E = 64000
B = 8
T = 12
S = B * T

NC = 2
NS = 16
L = 16
NW = NC * NS
ROWS_PER_W = 63
EB = 512
NEB = E // EB


def _build_adj_sc(src, dst):
    mesh = plsc.VectorSubcoreMesh(core_axis_name="c", subcore_axis_name="s")
    cp = pltpu.CompilerParams()
    if "needs_layout_passes" in pltpu.CompilerParams.__dataclass_fields__:
        cp = dataclasses.replace(cp, needs_layout_passes=False)

    @functools.partial(
        pl.kernel,
        out_type=jax.ShapeDtypeStruct((NW * ROWS_PER_W * N,), jnp.float32),
        mesh=mesh,
        compiler_params=cp,
        scratch_types=[
            pltpu.VMEM((ROWS_PER_W * N,), jnp.float32),
            pltpu.VMEM((EB,), jnp.int32),
            pltpu.VMEM((EB,), jnp.int32),
            pltpu.SemaphoreType.DMA,
        ],
    )
    def k(src_hbm, dst_hbm, a_hbm, a_loc, src_v, dst_v, sem):
        cid = lax.axis_index("c")
        sid = lax.axis_index("s")
        wid = sid * NC + cid
        base = wid * ROWS_PER_W

        zeros16 = jnp.zeros((L,), jnp.float32)
        ones16 = jnp.ones((L,), jnp.float32)
        lane = lax.iota(jnp.int32, L)

        @pl.loop(0, ROWS_PER_W * N, step=L)
        def _(cc):
            a_loc[pl.ds(cc, L)] = zeros16

        for kk in range((ROWS_PER_W + L - 1) // L):
            rows = lane + kk * L
            gcol = rows + base
            m = (rows < ROWS_PER_W) & (gcol < N)
            rows_c = jnp.minimum(rows, ROWS_PER_W - 1)
            flat = rows_c * N + jnp.minimum(gcol, N - 1)
            plsc.addupdate_scatter(a_loc, [flat], ones16, mask=m)

        @pl.loop(0, NEB)
        def _(blk):
            pltpu.sync_copy(src_hbm.at[pl.ds(blk * EB, EB)], src_v)
            pltpu.sync_copy(dst_hbm.at[pl.ds(blk * EB, EB)], dst_v)

            @pl.loop(0, EB, step=L)
            def _(e0):
                s16 = src_v[pl.ds(e0, L)]
                d16 = dst_v[pl.ds(e0, L)]
                row = d16 - base
                inr = (row >= 0) & (row < ROWS_PER_W)
                rowc = jnp.minimum(jnp.maximum(row, 0), ROWS_PER_W - 1)
                flat = rowc * N + s16
                for j in range(L):
                    mj = inr & (lane == j)
                    plsc.addupdate_scatter(a_loc, [flat], ones16, mask=mj)

        pltpu.async_copy(
            a_loc, a_hbm.at[pl.ds(wid * (ROWS_PER_W * N), ROWS_PER_W * N)], sem
        ).wait()

    return k(src, dst)


def _tc_body(a_ref, x_ref, w_ref, b_ref, o_ref, dis_ref):
    @pl.when(pl.program_id(0) == 0)
    def _():
        deg = jnp.sum(a_ref[...].astype(jnp.float32), axis=1, keepdims=True)
        dis_ref[...] = lax.rsqrt(deg)

    dis = dis_ref[...]
    xw = jnp.dot(
        x_ref[0].astype(jnp.bfloat16),
        w_ref[...].astype(jnp.bfloat16),
        preferred_element_type=jnp.float32,
    )
    z = (dis * xw).astype(jnp.bfloat16)
    y = jnp.dot(a_ref[...], z, preferred_element_type=jnp.float32)
    o_ref[0] = dis * y + b_ref[...]


def _gcn_tc(a_bf, x96, w, b2d):
    return pl.pallas_call(
        _tc_body,
        grid=(S,),
        in_specs=[
            pl.BlockSpec((N, N), lambda s: (0, 0)),
            pl.BlockSpec((1, N, F), lambda s: (s, 0, 0)),
            pl.BlockSpec((F, F), lambda s: (0, 0)),
            pl.BlockSpec((1, F), lambda s: (0, 0)),
        ],
        out_specs=pl.BlockSpec((1, N, F), lambda s: (s, 0, 0)),
        out_shape=jax.ShapeDtypeStruct((S, N, F), jnp.float32),
        scratch_shapes=[pltpu.VMEM((N, 1), jnp.float32)],
    )(a_bf, x96, w, b2d)


def kernel(x, edge_index, W, b):
    ei = edge_index.astype(jnp.int32)
    a_pad = _build_adj_sc(ei[0], ei[1])
    a_bf = a_pad.reshape(NW * ROWS_PER_W, N)[:N].astype(jnp.bfloat16)
    x96 = x.reshape(S, N, F)
    y = _gcn_tc(a_bf, x96, W, b.reshape(1, F))
    return y.reshape(B, T, N, F)


# speedup vs baseline: 190.8779x; 1.2955x over previous
import dataclasses
import functools

import jax
import jax.numpy as jnp
from jax import lax
from jax.experimental import pallas as pl
from jax.experimental.pallas import tpu as pltpu
from jax.experimental.pallas import tpu_sc as plsc

N = 2000
F = 128
E = 64000
B = 8
T = 12
S = B * T

NC = 2
NS = 16
L = 16
NW = NC * NS
ROWS_PER_W = 63
EB = 800
NEB = E // EB


def _build_adj_sc(src, dst):
    mesh = plsc.VectorSubcoreMesh(core_axis_name="c", subcore_axis_name="s")
    cp = pltpu.CompilerParams()
    if "needs_layout_passes" in pltpu.CompilerParams.__dataclass_fields__:
        cp = dataclasses.replace(cp, needs_layout_passes=False)

    @functools.partial(
        pl.kernel,
        out_type=jax.ShapeDtypeStruct((NW * ROWS_PER_W * N,), jnp.float32),
        mesh=mesh,
        compiler_params=cp,
        scratch_types=[
            pltpu.VMEM((ROWS_PER_W * N,), jnp.float32),
            pltpu.VMEM((EB,), jnp.int32),
            pltpu.VMEM((EB,), jnp.int32),
            pltpu.VMEM((EB,), jnp.int32),
            pltpu.VMEM((EB,), jnp.int32),
            pltpu.SemaphoreType.DMA,
            pltpu.SemaphoreType.DMA,
            pltpu.SemaphoreType.DMA,
        ],
    )
    def k(src_hbm, dst_hbm, a_hbm, a_loc, src_v0, src_v1, dst_v0, dst_v1,
          sem, sem0, sem1):
        cid = lax.axis_index("c")
        sid = lax.axis_index("s")
        wid = sid * NC + cid
        base = wid * ROWS_PER_W

        zeros16 = jnp.zeros((L,), jnp.float32)
        ones16 = jnp.ones((L,), jnp.float32)
        lane = lax.iota(jnp.int32, L)

        @pl.loop(0, ROWS_PER_W * N, step=L)
        def _(cc):
            a_loc[pl.ds(cc, L)] = zeros16

        for kk in range((ROWS_PER_W + L - 1) // L):
            rows = lane + kk * L
            gcol = rows + base
            m = (rows < ROWS_PER_W) & (gcol < N)
            rows_c = jnp.minimum(rows, ROWS_PER_W - 1)
            flat = rows_c * N + jnp.minimum(gcol, N - 1)
            plsc.addupdate_scatter(a_loc, [flat], ones16, mask=m)

        sems = (sem0, sem1)
        srcs = (src_v0, src_v1)
        dsts = (dst_v0, dst_v1)

        def start(blk, buf):
            pltpu.async_copy(src_hbm.at[pl.ds(blk * EB, EB)], srcs[buf],
                             sems[buf])
            pltpu.async_copy(dst_hbm.at[pl.ds(blk * EB, EB)], dsts[buf],
                             sems[buf])

        def wait(blk, buf):
            pltpu.make_async_copy(src_hbm.at[pl.ds(blk * EB, EB)],
                                  srcs[buf], sems[buf]).wait()
            pltpu.make_async_copy(dst_hbm.at[pl.ds(blk * EB, EB)],
                                  dsts[buf], sems[buf]).wait()

        def process(buf):
            @pl.loop(0, EB, step=L)
            def _(e0):
                s16 = srcs[buf][pl.ds(e0, L)]
                d16 = dsts[buf][pl.ds(e0, L)]
                row = d16 - base
                inr = (row >= 0) & (row < ROWS_PER_W)
                rowc = jnp.minimum(jnp.maximum(row, 0), ROWS_PER_W - 1)
                flat = rowc * N + s16
                for j in range(L):
                    mj = inr & (lane == j)
                    plsc.addupdate_scatter(a_loc, [flat], ones16, mask=mj)

        start(0, 0)

        @pl.loop(0, NEB, step=2)
        def _(blk):
            start(blk + 1, 1)
            wait(blk, 0)
            process(0)

            @pl.when(blk + 2 < NEB)
            def _():
                start(blk + 2, 0)

            wait(blk + 1, 1)
            process(1)

        pltpu.async_copy(
            a_loc, a_hbm.at[pl.ds(wid * (ROWS_PER_W * N), ROWS_PER_W * N)], sem
        ).wait()

    return k(src, dst)


def _tc_body(a_ref, x_ref, w_ref, b_ref, o_ref, dis_ref):
    @pl.when(pl.program_id(0) == 0)
    def _():
        deg = jnp.sum(a_ref[...].astype(jnp.float32), axis=1, keepdims=True)
        dis_ref[...] = lax.rsqrt(deg)

    dis = dis_ref[...]
    xw = jnp.dot(
        x_ref[0].astype(jnp.bfloat16),
        w_ref[...].astype(jnp.bfloat16),
        preferred_element_type=jnp.float32,
    )
    z = (dis * xw).astype(jnp.bfloat16)
    y = jnp.dot(a_ref[...], z, preferred_element_type=jnp.float32)
    o_ref[0] = dis * y + b_ref[...]


def _gcn_tc(a_bf, x96, w, b2d):
    return pl.pallas_call(
        _tc_body,
        grid=(S,),
        in_specs=[
            pl.BlockSpec((N, N), lambda s: (0, 0)),
            pl.BlockSpec((1, N, F), lambda s: (s, 0, 0)),
            pl.BlockSpec((F, F), lambda s: (0, 0)),
            pl.BlockSpec((1, F), lambda s: (0, 0)),
        ],
        out_specs=pl.BlockSpec((1, N, F), lambda s: (s, 0, 0)),
        out_shape=jax.ShapeDtypeStruct((S, N, F), jnp.float32),
        scratch_shapes=[pltpu.VMEM((N, 1), jnp.float32)],
    )(a_bf, x96, w, b2d)


def kernel(x, edge_index, W, b):
    ei = edge_index.astype(jnp.int32)
    a_pad = _build_adj_sc(ei[0], ei[1])
    a_bf = a_pad.reshape(NW * ROWS_PER_W, N)[:N].astype(jnp.bfloat16)
    x96 = x.reshape(S, N, F)
    y = _gcn_tc(a_bf, x96, W, b.reshape(1, F))
    return y.reshape(B, T, N, F)


# speedup vs baseline: 290.1807x; 1.5202x over previous
import dataclasses
import functools

import jax
import jax.numpy as jnp
from jax import lax
from jax.experimental import pallas as pl
from jax.experimental.pallas import tpu as pltpu
from jax.experimental.pallas import tpu_sc as plsc

N = 2000
F = 128
E = 64000
B = 8
T = 12
S = B * T

NC = 2
NS = 16
L = 16
NW = NC * NS
ROWS_PER_W = 63
EB = 800
NEB = E // EB


def _build_adj_sc(src, dst):
    mesh = plsc.VectorSubcoreMesh(core_axis_name="c", subcore_axis_name="s")
    cp = pltpu.CompilerParams()
    if "needs_layout_passes" in pltpu.CompilerParams.__dataclass_fields__:
        cp = dataclasses.replace(cp, needs_layout_passes=False)

    @functools.partial(
        pl.kernel,
        out_type=jax.ShapeDtypeStruct((NW * ROWS_PER_W * N,), jnp.float32),
        mesh=mesh,
        compiler_params=cp,
        scratch_types=[
            pltpu.VMEM((ROWS_PER_W * N,), jnp.float32),
            pltpu.VMEM((EB,), jnp.int32),
            pltpu.VMEM((EB,), jnp.int32),
            pltpu.VMEM((EB,), jnp.int32),
            pltpu.VMEM((EB,), jnp.int32),
            pltpu.SemaphoreType.DMA,
            pltpu.SemaphoreType.DMA,
            pltpu.SemaphoreType.DMA,
        ],
    )
    def k(src_hbm, dst_hbm, a_hbm, a_loc, src_v0, src_v1, dst_v0, dst_v1,
          sem, sem0, sem1):
        cid = lax.axis_index("c")
        sid = lax.axis_index("s")
        wid = sid * NC + cid
        base = wid * ROWS_PER_W

        zeros16 = jnp.zeros((L,), jnp.float32)
        ones16 = jnp.ones((L,), jnp.float32)
        lane = lax.iota(jnp.int32, L)

        @pl.loop(0, ROWS_PER_W * N, step=L)
        def _(cc):
            a_loc[pl.ds(cc, L)] = zeros16

        for kk in range((ROWS_PER_W + L - 1) // L):
            rows = lane + kk * L
            gcol = rows + base
            m = (rows < ROWS_PER_W) & (gcol < N)
            rows_c = jnp.minimum(rows, ROWS_PER_W - 1)
            flat = rows_c * N + jnp.minimum(gcol, N - 1)
            plsc.addupdate_scatter(a_loc, [flat], ones16, mask=m)

        sems = (sem0, sem1)
        srcs = (src_v0, src_v1)
        dsts = (dst_v0, dst_v1)

        def start(blk, buf):
            pltpu.async_copy(src_hbm.at[pl.ds(blk * EB, EB)], srcs[buf],
                             sems[buf])
            pltpu.async_copy(dst_hbm.at[pl.ds(blk * EB, EB)], dsts[buf],
                             sems[buf])

        def wait(blk, buf):
            pltpu.make_async_copy(src_hbm.at[pl.ds(blk * EB, EB)],
                                  srcs[buf], sems[buf]).wait()
            pltpu.make_async_copy(dst_hbm.at[pl.ds(blk * EB, EB)],
                                  dsts[buf], sems[buf]).wait()

        def process(buf):
            @pl.loop(0, EB, step=L)
            def _(e0):
                s16 = srcs[buf][pl.ds(e0, L)]
                d16 = dsts[buf][pl.ds(e0, L)]
                row = d16 - base
                inr = (row >= 0) & (row < ROWS_PER_W)
                rowc = jnp.minimum(jnp.maximum(row, 0), ROWS_PER_W - 1)
                flat = rowc * N + s16
                for j in range(L):
                    mj = inr & (lane == j)
                    plsc.addupdate_scatter(a_loc, [flat], ones16, mask=mj)

        start(0, 0)

        @pl.loop(0, NEB, step=2)
        def _(blk):
            start(blk + 1, 1)
            wait(blk, 0)
            process(0)

            @pl.when(blk + 2 < NEB)
            def _():
                start(blk + 2, 0)

            wait(blk + 1, 1)
            process(1)

        pltpu.async_copy(
            a_loc, a_hbm.at[pl.ds(wid * (ROWS_PER_W * N), ROWS_PER_W * N)], sem
        ).wait()

    return k(src, dst)


SB = 4


def _tc_body(a_ref, x_ref, w_ref, b_ref, o_ref, dis_ref):
    @pl.when(pl.program_id(0) == 0)
    def _():
        deg = jnp.sum(a_ref[...].astype(jnp.float32), axis=1, keepdims=True)
        dis_ref[...] = lax.rsqrt(deg)

    dis = dis_ref[...]
    xw = jnp.dot(
        x_ref[...].reshape(SB * N, F).astype(jnp.bfloat16),
        w_ref[...].astype(jnp.bfloat16),
        preferred_element_type=jnp.float32,
    )
    z4 = jnp.concatenate(
        [(dis * xw[i * N:(i + 1) * N]).astype(jnp.bfloat16) for i in range(SB)],
        axis=1,
    )
    y4 = jnp.dot(a_ref[...], z4, preferred_element_type=jnp.float32)
    for i in range(SB):
        o_ref[i] = dis * y4[:, i * F:(i + 1) * F] + b_ref[...]


def _gcn_tc(a_bf, x96, w, b2d):
    return pl.pallas_call(
        _tc_body,
        grid=(S // SB,),
        in_specs=[
            pl.BlockSpec((N, N), lambda s: (0, 0)),
            pl.BlockSpec((SB, N, F), lambda s: (s, 0, 0)),
            pl.BlockSpec((F, F), lambda s: (0, 0)),
            pl.BlockSpec((1, F), lambda s: (0, 0)),
        ],
        out_specs=pl.BlockSpec((SB, N, F), lambda s: (s, 0, 0)),
        out_shape=jax.ShapeDtypeStruct((S, N, F), jnp.float32),
        scratch_shapes=[pltpu.VMEM((N, 1), jnp.float32)],
    )(a_bf, x96, w, b2d)


def kernel(x, edge_index, W, b):
    ei = edge_index.astype(jnp.int32)
    a_pad = _build_adj_sc(ei[0], ei[1])
    a_bf = a_pad.reshape(NW * ROWS_PER_W, N)[:N].astype(jnp.bfloat16)
    x96 = x.reshape(S, N, F)
    y = _gcn_tc(a_bf, x96, W, b.reshape(1, F))
    return y.reshape(B, T, N, F)


# speedup vs baseline: 326.8909x; 1.1265x over previous
import dataclasses
import functools

import jax
import jax.numpy as jnp
from jax import lax
from jax.experimental import pallas as pl
from jax.experimental.pallas import tpu as pltpu
from jax.experimental.pallas import tpu_sc as plsc

N = 2000
F = 128
E = 64000
B = 8
T = 12
S = B * T

NC = 2
NS = 16
L = 16
NW = NC * NS
ROWS_PER_W = 63
EB = 800
NEB = E // EB


def _build_adj_sc(src, dst):
    mesh = plsc.VectorSubcoreMesh(core_axis_name="c", subcore_axis_name="s")
    cp = pltpu.CompilerParams()
    if "needs_layout_passes" in pltpu.CompilerParams.__dataclass_fields__:
        cp = dataclasses.replace(cp, needs_layout_passes=False)

    @functools.partial(
        pl.kernel,
        out_type=jax.ShapeDtypeStruct((NW * ROWS_PER_W * N,), jnp.float32),
        mesh=mesh,
        compiler_params=cp,
        scratch_types=[
            pltpu.VMEM((ROWS_PER_W * N,), jnp.float32),
            pltpu.VMEM((EB,), jnp.int32),
            pltpu.VMEM((EB,), jnp.int32),
            pltpu.VMEM((EB,), jnp.int32),
            pltpu.VMEM((EB,), jnp.int32),
            pltpu.SemaphoreType.DMA,
            pltpu.SemaphoreType.DMA,
            pltpu.SemaphoreType.DMA,
        ],
    )
    def k(src_hbm, dst_hbm, a_hbm, a_loc, src_v0, src_v1, dst_v0, dst_v1,
          sem, sem0, sem1):
        cid = lax.axis_index("c")
        sid = lax.axis_index("s")
        wid = sid * NC + cid
        base = wid * ROWS_PER_W

        zeros16 = jnp.zeros((L,), jnp.float32)
        ones16 = jnp.ones((L,), jnp.float32)
        lane = lax.iota(jnp.int32, L)

        @pl.loop(0, ROWS_PER_W * N, step=L)
        def _(cc):
            a_loc[pl.ds(cc, L)] = zeros16

        for kk in range((ROWS_PER_W + L - 1) // L):
            rows = lane + kk * L
            gcol = rows + base
            m = (rows < ROWS_PER_W) & (gcol < N)
            rows_c = jnp.minimum(rows, ROWS_PER_W - 1)
            flat = rows_c * N + jnp.minimum(gcol, N - 1)
            plsc.addupdate_scatter(a_loc, [flat], ones16, mask=m)

        sems = (sem0, sem1)
        srcs = (src_v0, src_v1)
        dsts = (dst_v0, dst_v1)

        def start(blk, buf):
            pltpu.async_copy(src_hbm.at[pl.ds(blk * EB, EB)], srcs[buf],
                             sems[buf])
            pltpu.async_copy(dst_hbm.at[pl.ds(blk * EB, EB)], dsts[buf],
                             sems[buf])

        def wait(blk, buf):
            pltpu.make_async_copy(src_hbm.at[pl.ds(blk * EB, EB)],
                                  srcs[buf], sems[buf]).wait()
            pltpu.make_async_copy(dst_hbm.at[pl.ds(blk * EB, EB)],
                                  dsts[buf], sems[buf]).wait()

        def process(buf):
            @pl.loop(0, EB, step=L)
            def _(e0):
                s16 = srcs[buf][pl.ds(e0, L)]
                d16 = dsts[buf][pl.ds(e0, L)]
                row = d16 - base
                inr = (row >= 0) & (row < ROWS_PER_W)
                rowc = jnp.minimum(jnp.maximum(row, 0), ROWS_PER_W - 1)
                flat = rowc * N + s16
                plsc.addupdate_scatter(a_loc, [flat], ones16, mask=inr)

        start(0, 0)

        @pl.loop(0, NEB, step=2)
        def _(blk):
            start(blk + 1, 1)
            wait(blk, 0)
            process(0)

            @pl.when(blk + 2 < NEB)
            def _():
                start(blk + 2, 0)

            wait(blk + 1, 1)
            process(1)

        pltpu.async_copy(
            a_loc, a_hbm.at[pl.ds(wid * (ROWS_PER_W * N), ROWS_PER_W * N)], sem
        ).wait()

    return k(src, dst)


SB = 4


def _tc_body(a_ref, x_ref, w_ref, b_ref, o_ref, as_ref):
    @pl.when(pl.program_id(0) == 0)
    def _():
        a = a_ref[...].astype(jnp.float32)
        dis = lax.rsqrt(jnp.sum(a, axis=1, keepdims=True))
        as_ref[...] = (a * dis * dis.reshape(1, N)).astype(jnp.bfloat16)

    xw = jnp.dot(
        x_ref[...].reshape(SB * N, F),
        w_ref[...],
        preferred_element_type=jnp.float32,
    )
    z4 = jnp.concatenate(
        [xw[i * N:(i + 1) * N].astype(jnp.bfloat16) for i in range(SB)],
        axis=1,
    )
    y4 = jnp.dot(as_ref[...], z4, preferred_element_type=jnp.float32)
    for i in range(SB):
        o_ref[i] = y4[:, i * F:(i + 1) * F] + b_ref[...]


def _gcn_tc(a_bf, x96, w, b2d):
    return pl.pallas_call(
        _tc_body,
        grid=(S // SB,),
        in_specs=[
            pl.BlockSpec((N, N), lambda s: (0, 0)),
            pl.BlockSpec((SB, N, F), lambda s: (s, 0, 0)),
            pl.BlockSpec((F, F), lambda s: (0, 0)),
            pl.BlockSpec((1, F), lambda s: (0, 0)),
        ],
        out_specs=pl.BlockSpec((SB, N, F), lambda s: (s, 0, 0)),
        out_shape=jax.ShapeDtypeStruct((S, N, F), jnp.float32),
        scratch_shapes=[pltpu.VMEM((N, N), jnp.bfloat16)],
    )(a_bf, x96, w, b2d)


def kernel(x, edge_index, W, b):
    ei = edge_index.astype(jnp.int32)
    a_pad = _build_adj_sc(ei[0], ei[1])
    a_bf = a_pad.reshape(NW * ROWS_PER_W, N)[:N].astype(jnp.bfloat16)
    x96 = x.reshape(S, N, F).astype(jnp.bfloat16)
    y = _gcn_tc(a_bf, x96, W.astype(jnp.bfloat16), b.reshape(1, F))
    return y.reshape(B, T, N, F)
